# baseline (speedup 1.0000x reference)
import jax
import jax.numpy as jnp
from jax.experimental import pallas as pl
from jax.experimental.pallas import tpu as pltpu


def _fused_block_kernel(x_ref, w_ref, be_ref, i_ref, w1_ref, b1_ref, w2_ref,
                        g_ref, gb2_ref, o_ref, s_ref):
    C, H, W = x_ref.shape
    K = w_ref.shape[0]
    P = K // 2
    HW = H * W
    WL = 8

    x2d = x_ref[...].reshape(C, HW)

    xt = jax.lax.dot_general(x2d, i_ref[...], (((0,), (0,)), ((), ())),
                             preferred_element_type=jnp.float32)

    s_ref[0:P, :, :] = jnp.zeros((P, s_ref.shape[1], C), jnp.float32)
    s_ref[P + H:, :, :] = jnp.zeros((s_ref.shape[0] - P - H, s_ref.shape[1], C),
                                    jnp.float32)
    s_ref[:, 0:WL, :] = jnp.zeros((s_ref.shape[0], WL, C), jnp.float32)
    s_ref[:, WL + W:, :] = jnp.zeros(
        (s_ref.shape[0], s_ref.shape[1] - WL - W, C), jnp.float32)
    s_ref[P:P + H, WL:WL + W, :] = xt.reshape(H, W, C)

    acc = jnp.broadcast_to(be_ref[...].reshape(1, 1, C), (H, W, C))
    for kh in range(K):
        for kw in range(K):
            win = s_ref[kh:kh + H, WL - P + kw:WL - P + kw + W, :]
            wv = w_ref[kh, kw:kw + 1, :].reshape(1, 1, C)
            acc = acc + win * wv

    t = acc.reshape(HW, C).astype(jnp.bfloat16)
    h = jnp.dot(t, w1_ref[...], preferred_element_type=jnp.float32)
    h = h + b1_ref[...]
    h = h * jnp.clip(h + 3.0, 0.0, 6.0) * (1.0 / 6.0)
    y = jax.lax.dot_general(w2_ref[...], h.astype(jnp.bfloat16),
                            (((0,), (1,)), ((), ())),
                            preferred_element_type=jnp.float32)

    out = x2d + g_ref[...] * y + gb2_ref[...]
    o_ref[...] = out.reshape(C, H, W).astype(o_ref.dtype)


def kernel(x, w_dw, b_dw, bn_w, bn_b, bn_mean, bn_var, w1, b1, w2, b2, gamma):
    N, C, H, W = x.shape
    K = w_dw.shape[-1]
    P = K // 2
    CE = w1.shape[1]
    Hp = H + 2 * P
    Wp = W + 16

    s = bn_w * jax.lax.rsqrt(bn_var + 1e-5)
    w_eff = jnp.transpose(w_dw[:, 0, :, :], (1, 2, 0)) * s
    b_eff = ((b_dw - bn_mean) * s + bn_b).reshape(1, C)

    eye = jnp.eye(C, dtype=jnp.float32)
    g_col = gamma.reshape(C, 1)
    gb2_col = (gamma * b2).reshape(C, 1)

    out = pl.pallas_call(
        _fused_block_kernel,
        out_shape=jax.ShapeDtypeStruct((N, C, H, W), x.dtype),
        grid=(N,),
        in_specs=[
            pl.BlockSpec((None, C, H, W), lambda n: (n, 0, 0, 0)),
            pl.BlockSpec((K, K, C), lambda n: (0, 0, 0)),
            pl.BlockSpec((1, C), lambda n: (0, 0)),
            pl.BlockSpec((C, C), lambda n: (0, 0)),
            pl.BlockSpec((C, CE), lambda n: (0, 0)),
            pl.BlockSpec((1, CE), lambda n: (0, 0)),
            pl.BlockSpec((CE, C), lambda n: (0, 0)),
            pl.BlockSpec((C, 1), lambda n: (0, 0)),
            pl.BlockSpec((C, 1), lambda n: (0, 0)),
        ],
        out_specs=pl.BlockSpec((None, C, H, W), lambda n: (n, 0, 0, 0)),
        scratch_shapes=[pltpu.VMEM((Hp, Wp, C), jnp.float32)],
        compiler_params=pltpu.CompilerParams(dimension_semantics=("parallel",)),
    )(x, w_eff, b_eff, eye, w1.astype(jnp.bfloat16), b1.reshape(1, CE),
      w2.astype(jnp.bfloat16), g_col, gb2_col)

    return out


# speedup vs baseline: 1.3607x; 1.3607x over previous
import functools

import jax
import jax.numpy as jnp
from jax.experimental import pallas as pl
from jax.experimental.pallas import tpu as pltpu


def _fused_block_kernel(x_ref, w_ref, be_ref, i_ref, w1_ref, b1_ref, w2_ref,
                        g_ref, gb2_ref, o_ref, s_ref, *, H, W):
    C = x_ref.shape[0]
    K = w_ref.shape[0]
    P = K // 2
    HW = H * W
    WL = 8

    x2d = x_ref[...]

    xt = jax.lax.dot_general(x2d, i_ref[...], (((0,), (0,)), ((), ())),
                             preferred_element_type=jnp.float32)

    s_ref[0:P, :, :] = jnp.zeros((P, s_ref.shape[1], C), jnp.float32)
    s_ref[P + H:, :, :] = jnp.zeros((s_ref.shape[0] - P - H, s_ref.shape[1], C),
                                    jnp.float32)
    s_ref[:, 0:WL, :] = jnp.zeros((s_ref.shape[0], WL, C), jnp.float32)
    s_ref[:, WL + W:, :] = jnp.zeros(
        (s_ref.shape[0], s_ref.shape[1] - WL - W, C), jnp.float32)
    s_ref[P:P + H, WL:WL + W, :] = xt.reshape(H, W, C)

    acc = jnp.broadcast_to(be_ref[...].reshape(1, 1, C), (H, W, C))
    for kh in range(K):
        for kw in range(K):
            win = s_ref[kh:kh + H, WL - P + kw:WL - P + kw + W, :]
            wv = w_ref[kh, kw:kw + 1, :].reshape(1, 1, C)
            acc = acc + win * wv

    t = acc.reshape(HW, C).astype(jnp.bfloat16)
    h = jnp.dot(t, w1_ref[...], preferred_element_type=jnp.float32)
    h = h + b1_ref[...]
    h = h * jnp.clip(h + 3.0, 0.0, 6.0) * (1.0 / 6.0)
    y = jax.lax.dot_general(w2_ref[...], h.astype(jnp.bfloat16),
                            (((0,), (1,)), ((), ())),
                            preferred_element_type=jnp.float32)

    out = x2d + g_ref[...] * y + gb2_ref[...]
    o_ref[...] = out.astype(o_ref.dtype)


def kernel(x, w_dw, b_dw, bn_w, bn_b, bn_mean, bn_var, w1, b1, w2, b2, gamma):
    N, C, H, W = x.shape
    K = w_dw.shape[-1]
    P = K // 2
    CE = w1.shape[1]
    Hp = H + 2 * P
    Wp = W + 16

    s = bn_w * jax.lax.rsqrt(bn_var + 1e-5)
    w_eff = jnp.transpose(w_dw[:, 0, :, :], (1, 2, 0)) * s
    b_eff = ((b_dw - bn_mean) * s + bn_b).reshape(1, C)

    eye = jnp.eye(C, dtype=jnp.float32)
    g_col = gamma.reshape(C, 1)
    gb2_col = (gamma * b2).reshape(C, 1)

    body = functools.partial(_fused_block_kernel, H=H, W=W)
    x3d = x.reshape(N, C, H * W)

    out = pl.pallas_call(
        body,
        out_shape=jax.ShapeDtypeStruct((N, C, H * W), x.dtype),
        grid=(N,),
        in_specs=[
            pl.BlockSpec((None, C, H * W), lambda n: (n, 0, 0)),
            pl.BlockSpec((K, K, C), lambda n: (0, 0, 0)),
            pl.BlockSpec((1, C), lambda n: (0, 0)),
            pl.BlockSpec((C, C), lambda n: (0, 0)),
            pl.BlockSpec((C, CE), lambda n: (0, 0)),
            pl.BlockSpec((1, CE), lambda n: (0, 0)),
            pl.BlockSpec((CE, C), lambda n: (0, 0)),
            pl.BlockSpec((C, 1), lambda n: (0, 0)),
            pl.BlockSpec((C, 1), lambda n: (0, 0)),
        ],
        out_specs=pl.BlockSpec((None, C, H * W), lambda n: (n, 0, 0)),
        scratch_shapes=[pltpu.VMEM((Hp, Wp, C), jnp.float32)],
        compiler_params=pltpu.CompilerParams(dimension_semantics=("parallel",)),
    )(x3d, w_eff, b_eff, eye, w1.astype(jnp.bfloat16), b1.reshape(1, CE),
      w2.astype(jnp.bfloat16), g_col, gb2_col)

    return out.reshape(N, C, H, W)
